# baseline (speedup 1.0000x reference)
import functools

import numpy as np
import jax
import jax.numpy as jnp
from jax import lax
from jax.experimental import pallas as pl
from jax.experimental.pallas import tpu as pltpu
from jax.experimental.pallas import tpu_sc as plsc

_N = 1048576
_C = 128
_NBINS = 15
_NC = 2
_NS = 16
_NW = _NC * _NS
_ROWS_PER_W = _N // _NW
_CHUNK = 128
_NCHUNK = _ROWS_PER_W // _CHUNK
_GROUPS = _CHUNK // 16

_BOUNDS = np.linspace(0.0, 1.0, _NBINS + 1).astype(np.float32)


def _ece_body(sm_hbm, lbl_hbm, out_hbm,
              rows0, rows1, lbl0, lbl1, acc,
              sem_r0, sem_r1, sem_l0, sem_l1):
    wid = lax.axis_index("s") * _NC + lax.axis_index("c")
    base = wid * _ROWS_PER_W

    rows_bufs = (rows0, rows1)
    lbl_bufs = (lbl0, lbl1)
    sem_r = (sem_r0, sem_r1)
    sem_l = (sem_l0, sem_l1)

    lanes = lax.iota(jnp.int32, 16)
    zero16 = jnp.zeros((16,), jnp.float32)

    for t in range(48):
        acc[pl.ds(t * 16, 16)] = zero16

    def start(cur, b):
        r0 = base + cur * _CHUNK
        pltpu.async_copy(sm_hbm.at[pl.ds(r0, _CHUNK), :],
                         rows_bufs[b].at[:, pl.ds(0, _C)], sem_r[b])
        pltpu.async_copy(lbl_hbm.at[pl.ds(r0, _CHUNK)], lbl_bufs[b], sem_l[b])

    def wait(cur, b):
        r0 = base + cur * _CHUNK
        pltpu.make_async_copy(
            sm_hbm.at[pl.ds(r0, _CHUNK), :],
            rows_bufs[b].at[:, pl.ds(0, _C)], sem_r[b]).wait()
        pltpu.make_async_copy(
            lbl_hbm.at[pl.ds(r0, _CHUNK)], lbl_bufs[b], sem_l[b]).wait()

    nblk = _C // 16

    def compute(b):
        rows = rows_bufs[b]
        lblv = lbl_bufs[b]

        def gbody(g, _):
            rowvec = g * 16 + lanes

            m_blks = []
            for bb in range(nblk):
                mb = jnp.full((16,), -1.0, jnp.float32)
                for u in range(16):
                    cj = jnp.full((16,), bb * 16 + u, jnp.int32)
                    v = plsc.load_gather(rows, [rowvec, cj])
                    mb = jnp.maximum(mb, v)
                m_blks.append(mb)

            t1 = [jnp.maximum(m_blks[2 * i], m_blks[2 * i + 1]) for i in range(4)]
            t2 = [jnp.maximum(t1[0], t1[1]), jnp.maximum(t1[2], t1[3])]
            m = jnp.maximum(t2[0], t2[1])

            amb = jnp.zeros((16,), jnp.int32)
            for bb in range(nblk - 1, 0, -1):
                amb = jnp.where(m_blks[bb] == m,
                                jnp.full((16,), bb * 16, jnp.int32), amb)

            am = amb
            colscan = amb + 15
            for _u in range(16):
                v = plsc.load_gather(rows, [rowvec, colscan])
                am = jnp.where(v == m, colscan, am)
                colscan = colscan - 1

            lbl = lblv[pl.ds(g * 16, 16)]
            accv = jnp.where(am == lbl, 1.0, 0.0).astype(jnp.float32)

            t = jnp.zeros((16,), jnp.int32)
            for i in range(_NBINS):
                t = t + jnp.where(m > _BOUNDS[i], 1, 0).astype(jnp.int32)
            valid = (t > 0) & (m <= _BOUNDS[_NBINS])
            validf = jnp.where(valid, 1.0, 0.0).astype(jnp.float32)
            binv = jnp.maximum(t - 1, 0)

            slot = binv * 16 + lanes
            plsc.addupdate_scatter(acc, [slot], validf)
            plsc.addupdate_scatter(acc, [slot + 256], m * validf)
            plsc.addupdate_scatter(acc, [slot + 512], accv * validf)
            return 0

        lax.fori_loop(0, _GROUPS, gbody, 0)

    start(0, 0)
    start(1, 1)

    def outer(it, _):
        i = it * 2
        for b in range(2):
            cur = i + b
            wait(cur, b)
            compute(b)

            @pl.when(cur + 2 < _NCHUNK)
            def _prefetch():
                start(cur + 2, b)

        return 0

    lax.fori_loop(0, _NCHUNK // 2, outer, 0)

    pltpu.sync_copy(acc, out_hbm.at[wid])


_ece_partials = functools.partial(
    pl.kernel,
    out_type=jax.ShapeDtypeStruct((_NW, 768), jnp.float32),
    mesh=plsc.VectorSubcoreMesh(core_axis_name="c", subcore_axis_name="s"),
    compiler_params=pltpu.CompilerParams(needs_layout_passes=False),
    scratch_types=[
        pltpu.VMEM((_CHUNK, _C + 1), jnp.float32),
        pltpu.VMEM((_CHUNK, _C + 1), jnp.float32),
        pltpu.VMEM((_CHUNK,), jnp.int32),
        pltpu.VMEM((_CHUNK,), jnp.int32),
        pltpu.VMEM((768,), jnp.float32),
        pltpu.SemaphoreType.DMA,
        pltpu.SemaphoreType.DMA,
        pltpu.SemaphoreType.DMA,
        pltpu.SemaphoreType.DMA,
    ],
)(_ece_body)


def kernel(softmaxes, labels):
    parts = _ece_partials(softmaxes, labels)
    s = jnp.sum(parts, axis=0).reshape(48, 16).sum(axis=1)
    cnt = s[0:_NBINS]
    sum_conf = s[16:16 + _NBINS]
    sum_acc = s[32:32 + _NBINS]
    prop = cnt / _N
    safe = jnp.maximum(cnt, 1.0)
    contrib = jnp.abs(sum_conf / safe - sum_acc / safe) * prop
    ece = jnp.sum(jnp.where(prop > 0.0, contrib, 0.0))
    return ece.reshape(1)


# speedup vs baseline: 5.9430x; 5.9430x over previous
import functools

import numpy as np
import jax
import jax.numpy as jnp
from jax import lax
from jax.experimental import pallas as pl
from jax.experimental.pallas import tpu as pltpu
from jax.experimental.pallas import tpu_sc as plsc

_N = 1048576
_C = 128
_NBINS = 15
_NC = 2
_NS = 16
_NW = _NC * _NS
_ROWS_PER_W = _N // _NW
_CHUNK = 256
_NCHUNK = _ROWS_PER_W // _CHUNK
_GROUPS = _CHUNK // 16

_BOUNDS = np.linspace(0.0, 1.0, _NBINS + 1).astype(np.float32)


def _ece_body(sm_hbm, lbl_hbm, out_hbm,
              rows0, rows1, lbl0, lbl1, acc,
              sem_r0, sem_r1, sem_l0, sem_l1):
    wid = lax.axis_index("s") * _NC + lax.axis_index("c")
    base = wid * _ROWS_PER_W

    rows_bufs = (rows0, rows1)
    lbl_bufs = (lbl0, lbl1)
    sem_r = (sem_r0, sem_r1)
    sem_l = (sem_l0, sem_l1)

    lanes = lax.iota(jnp.int32, 16)
    zero16 = jnp.zeros((16,), jnp.float32)

    for t in range(48):
        acc[pl.ds(t * 16, 16)] = zero16

    def start(cur, b):
        r0 = base + cur * _CHUNK
        pltpu.async_copy(sm_hbm.at[pl.ds(r0 * _C, _CHUNK * _C)], rows_bufs[b],
                         sem_r[b])
        pltpu.async_copy(lbl_hbm.at[pl.ds(r0, _CHUNK)], lbl_bufs[b], sem_l[b])

    def wait(cur, b):
        r0 = base + cur * _CHUNK
        pltpu.make_async_copy(
            sm_hbm.at[pl.ds(r0 * _C, _CHUNK * _C)], rows_bufs[b], sem_r[b]).wait()
        pltpu.make_async_copy(
            lbl_hbm.at[pl.ds(r0, _CHUNK)], lbl_bufs[b], sem_l[b]).wait()

    nblk = _C // 16
    idx_pat = [lanes * _C + ((lanes + u) & 15) for u in range(16)]

    def compute(b):
        rows = rows_bufs[b]
        lblv = lbl_bufs[b]

        def gbody(g, _):
            gbase = g * (16 * _C)

            m_blks = []
            for bb in range(nblk):
                slb = rows.at[pl.ds(gbase + bb * 16, 15 * _C + 16)]
                mb = jnp.full((16,), -1.0, jnp.float32)
                for u in range(16):
                    v = plsc.load_gather(slb, [idx_pat[u]])
                    mb = jnp.maximum(mb, v)
                m_blks.append(mb)

            t1 = [jnp.maximum(m_blks[2 * i], m_blks[2 * i + 1]) for i in range(4)]
            t2 = [jnp.maximum(t1[0], t1[1]), jnp.maximum(t1[2], t1[3])]
            m = jnp.maximum(t2[0], t2[1])

            amb = jnp.zeros((16,), jnp.int32)
            for bb in range(nblk - 1, 0, -1):
                amb = jnp.where(m_blks[bb] == m,
                                jnp.full((16,), bb * 16, jnp.int32), amb)

            sl = rows.at[pl.ds(gbase, 16 * _C)]
            am = amb
            for u in range(15, -1, -1):
                idxfull = idx_pat[u] + amb
                v = plsc.load_gather(sl, [idxfull])
                am = jnp.where(v == m, idxfull & 127, am)

            lbl = lblv[pl.ds(g * 16, 16)]
            accv = jnp.where(am == lbl, 1.0, 0.0).astype(jnp.float32)

            t = jnp.zeros((16,), jnp.int32)
            for i in range(_NBINS):
                t = t + jnp.where(m > _BOUNDS[i], 1, 0).astype(jnp.int32)
            valid = (t > 0) & (m <= _BOUNDS[_NBINS])
            validf = jnp.where(valid, 1.0, 0.0).astype(jnp.float32)
            binv = jnp.maximum(t - 1, 0)

            slot = binv * 16 + lanes
            plsc.addupdate_scatter(acc, [slot], validf)
            plsc.addupdate_scatter(acc, [slot + 256], m * validf)
            plsc.addupdate_scatter(acc, [slot + 512], accv * validf)
            return 0

        lax.fori_loop(0, _GROUPS, gbody, 0)

    start(0, 0)
    start(1, 1)

    def outer(it, _):
        i = it * 2
        for b in range(2):
            cur = i + b
            wait(cur, b)
            compute(b)

            @pl.when(cur + 2 < _NCHUNK)
            def _prefetch():
                start(cur + 2, b)

        return 0

    lax.fori_loop(0, _NCHUNK // 2, outer, 0)

    pltpu.sync_copy(acc, out_hbm.at[wid])


_ece_partials = functools.partial(
    pl.kernel,
    out_type=jax.ShapeDtypeStruct((_NW, 768), jnp.float32),
    mesh=plsc.VectorSubcoreMesh(core_axis_name="c", subcore_axis_name="s"),
    compiler_params=pltpu.CompilerParams(needs_layout_passes=False),
    scratch_types=[
        pltpu.VMEM((_CHUNK * _C,), jnp.float32),
        pltpu.VMEM((_CHUNK * _C,), jnp.float32),
        pltpu.VMEM((_CHUNK,), jnp.int32),
        pltpu.VMEM((_CHUNK,), jnp.int32),
        pltpu.VMEM((768,), jnp.float32),
        pltpu.SemaphoreType.DMA,
        pltpu.SemaphoreType.DMA,
        pltpu.SemaphoreType.DMA,
        pltpu.SemaphoreType.DMA,
    ],
)(_ece_body)


def kernel(softmaxes, labels):
    parts = _ece_partials(softmaxes.reshape(-1), labels)
    s = jnp.sum(parts, axis=0).reshape(48, 16).sum(axis=1)
    cnt = s[0:_NBINS]
    sum_conf = s[16:16 + _NBINS]
    sum_acc = s[32:32 + _NBINS]
    prop = cnt / _N
    safe = jnp.maximum(cnt, 1.0)
    contrib = jnp.abs(sum_conf / safe - sum_acc / safe) * prop
    ece = jnp.sum(jnp.where(prop > 0.0, contrib, 0.0))
    return ece.reshape(1)
